# baseline (speedup 1.0000x reference)
import jax
import jax.numpy as jnp
from jax import lax
from jax.experimental import pallas as pl
from jax.experimental.pallas import tpu as pltpu

NE = 512
ED = 256
TM = 8192


def _vq_body(z_ref, e_ref, o_ref, e2_scr, e2x_scr, ebf_scr):
    Z = z_ref[...]

    @pl.when(pl.program_id(0) == 0)
    def _init():
        E = e_ref[...]
        ones = jnp.ones((1, ED), dtype=jnp.float32)
        e2_scr[...] = lax.dot_general(ones, E * E, (((1,), (1,)), ((), ())),
                                      precision=lax.Precision.HIGHEST)
        e2x_scr[...] = E + E
        ebf_scr[...] = E.astype(jnp.bfloat16)

    M2 = lax.dot_general(Z, e2x_scr[...], (((1,), (1,)), ((), ())))
    e2 = e2_scr[...]
    z2 = jnp.sum(Z * Z, axis=1, keepdims=True)
    D = (z2 + e2) - M2
    dmin = jnp.min(D, axis=1, keepdims=True)
    onehot = (D == dmin).astype(jnp.bfloat16)
    o_ref[...] = lax.dot_general(onehot, ebf_scr[...], (((1,), (0,)), ((), ())),
                                 preferred_element_type=jnp.float32)


def kernel(x, e):
    B, C, H, W = x.shape
    T = B * H * W
    z = jnp.transpose(x, (0, 2, 3, 1)).reshape(T, C)
    out = pl.pallas_call(
        _vq_body,
        grid=(T // TM,),
        in_specs=[
            pl.BlockSpec((TM, C), lambda i: (i, 0)),
            pl.BlockSpec((NE, C), lambda i: (0, 0)),
        ],
        out_specs=pl.BlockSpec((TM, C), lambda i: (i, 0)),
        out_shape=jax.ShapeDtypeStruct((T, C), jnp.float32),
        scratch_shapes=[pltpu.VMEM((1, NE), jnp.float32),
                        pltpu.VMEM((NE, ED), jnp.float32),
                        pltpu.VMEM((NE, ED), jnp.bfloat16)],
    )(z, e)
    return jnp.transpose(out.reshape(B, H, W, C), (0, 3, 1, 2))


# speedup vs baseline: 1.2230x; 1.2230x over previous
import jax
import jax.numpy as jnp
from jax import lax
from jax.experimental import pallas as pl
from jax.experimental.pallas import tpu as pltpu

NE = 512
ED = 256
TM = 8192


def _vq_body(z_ref, e_ref, o_ref, e2_scr, e2x_scr, ebf_scr):
    Z = z_ref[...]

    @pl.when(pl.program_id(0) == 0)
    def _init():
        E = e_ref[...]
        ones = jnp.ones((1, ED), dtype=jnp.float32)
        e2_scr[...] = lax.dot_general(ones, E * E, (((1,), (1,)), ((), ())),
                                      precision=lax.Precision.HIGHEST)
        e2x_scr[...] = E + E
        ebf_scr[...] = E.astype(jnp.bfloat16)

    e2 = e2_scr[...]
    e2x = e2x_scr[...]
    ebf = ebf_scr[...]
    half = TM // 2
    for h in range(2):
        Zh = Z[h * half:(h + 1) * half, :]
        M2 = lax.dot_general(Zh, e2x, (((1,), (1,)), ((), ())))
        z2 = jnp.sum(Zh * Zh, axis=1, keepdims=True)
        D = (z2 + e2) - M2
        dmin = jnp.min(D, axis=1, keepdims=True)
        onehot = (D == dmin).astype(jnp.bfloat16)
        o_ref[h * half:(h + 1) * half, :] = lax.dot_general(
            onehot, ebf, (((1,), (0,)), ((), ())),
            preferred_element_type=jnp.float32)


def kernel(x, e):
    B, C, H, W = x.shape
    T = B * H * W
    z = jnp.transpose(x, (0, 2, 3, 1)).reshape(T, C)
    out = pl.pallas_call(
        _vq_body,
        grid=(T // TM,),
        in_specs=[
            pl.BlockSpec((TM, C), lambda i: (i, 0)),
            pl.BlockSpec((NE, C), lambda i: (0, 0)),
        ],
        out_specs=pl.BlockSpec((TM, C), lambda i: (i, 0)),
        out_shape=jax.ShapeDtypeStruct((T, C), jnp.float32),
        scratch_shapes=[pltpu.VMEM((1, NE), jnp.float32),
                        pltpu.VMEM((NE, ED), jnp.float32),
                        pltpu.VMEM((NE, ED), jnp.bfloat16)],
    )(z, e)
    return jnp.transpose(out.reshape(B, H, W, C), (0, 3, 1, 2))


# speedup vs baseline: 1.3665x; 1.1174x over previous
import jax
import jax.numpy as jnp
from jax import lax
from jax.experimental import pallas as pl
from jax.experimental.pallas import tpu as pltpu

NE = 512
ED = 256
TM = 8192


def _vq_body(z_ref, e_ref, o_ref, e2_scr, e2x_scr, ebf_scr):
    Z = z_ref[...]

    @pl.when(pl.program_id(0) == 0)
    def _init():
        E = e_ref[...]
        ones = jnp.ones((1, ED), dtype=jnp.float32)
        e2_scr[...] = lax.dot_general(ones, E * E, (((1,), (1,)), ((), ())),
                                      precision=lax.Precision.HIGHEST)
        e2x_scr[...] = E + E
        ebf_scr[...] = E.astype(jnp.bfloat16)

    e2 = e2_scr[...]
    e2x = e2x_scr[...]
    ebf = ebf_scr[...]
    half = TM // 4
    for h in range(4):
        Zh = Z[h * half:(h + 1) * half, :]
        M2 = lax.dot_general(Zh, e2x, (((1,), (1,)), ((), ())))
        z2 = jnp.sum(Zh * Zh, axis=1, keepdims=True)
        D = (z2 + e2) - M2
        dmin = jnp.min(D, axis=1, keepdims=True)
        onehot = (D == dmin).astype(jnp.bfloat16)
        o_ref[h * half:(h + 1) * half, :] = lax.dot_general(
            onehot, ebf, (((1,), (0,)), ((), ())),
            preferred_element_type=jnp.float32)


def kernel(x, e):
    B, C, H, W = x.shape
    T = B * H * W
    z = jnp.transpose(x, (0, 2, 3, 1)).reshape(T, C)
    out = pl.pallas_call(
        _vq_body,
        grid=(T // TM,),
        in_specs=[
            pl.BlockSpec((TM, C), lambda i: (i, 0)),
            pl.BlockSpec((NE, C), lambda i: (0, 0)),
        ],
        out_specs=pl.BlockSpec((TM, C), lambda i: (i, 0)),
        out_shape=jax.ShapeDtypeStruct((T, C), jnp.float32),
        scratch_shapes=[pltpu.VMEM((1, NE), jnp.float32),
                        pltpu.VMEM((NE, ED), jnp.float32),
                        pltpu.VMEM((NE, ED), jnp.bfloat16)],
    )(z, e)
    return jnp.transpose(out.reshape(B, H, W, C), (0, 3, 1, 2))
